# baseline (speedup 1.0000x reference)
import jax
import jax.numpy as jnp
from jax.experimental import pallas as pl
from jax.experimental.pallas import tpu as pltpu

_EPSILON = 1e-4
_N_PAIRS = 4


def _sinkhorn_body(s_ref, o_ref):
    s0 = s_ref[...]
    m = jnp.sum(s0, axis=1, keepdims=True)
    v = 1.0 / m
    r = jnp.sum(s0 * v, axis=2, keepdims=True)
    w = r + _EPSILON
    u = 1.0 / w
    for _ in range(_N_PAIRS - 1):
        m = jax.lax.dot_general(u, s0, (((1,), (1,)), ((0,), (0,))),
                                preferred_element_type=jnp.float32)
        v = 1.0 / m
        r = jnp.sum(s0 * v, axis=2, keepdims=True)
        w = r + _EPSILON * w
        u = 1.0 / w
    m = jax.lax.dot_general(u, s0, (((1,), (1,)), ((0,), (0,))),
                            preferred_element_type=jnp.float32)
    v = 1.0 / m
    t = s0 * v
    r = jnp.sum(t, axis=2, keepdims=True)
    w = r + _EPSILON * w
    u = 1.0 / w
    o_ref[...] = t * u


def kernel(s):
    b, n1, n2 = s.shape
    return pl.pallas_call(
        _sinkhorn_body,
        grid=(b // 8,),
        in_specs=[pl.BlockSpec((8, n1, n2), lambda i: (i, 0, 0))],
        out_specs=pl.BlockSpec((8, n1, n2), lambda i: (i, 0, 0)),
        out_shape=jax.ShapeDtypeStruct(s.shape, s.dtype),
        compiler_params=pltpu.CompilerParams(
            dimension_semantics=("parallel",),
        ),
    )(s)
